# Initial kernel scaffold; baseline (speedup 1.0000x reference)
import jax
import jax.numpy as jnp
from jax.experimental import pallas as pl


def kernel(x_stay, x_code, ei_sc, ei_cs, W_in_stay, b_in_stay, W_in_code, b_in_code, Wl0_sc, bl0_sc, Wr0_sc, br0_sc, Wl0_cs, bl0_cs, Wr0_cs, br0_cs, g0, be0, Wl1_sc, bl1_sc, Wr1_sc, br1_sc, Wl1_cs, bl1_cs, Wr1_cs, br1_cs, g1, be1, Wc, bc):
    raise NotImplementedError("write your pallas kernel here")


# speedup vs baseline: 1.4588x; 1.4588x over previous
import jax
import jax.numpy as jnp
from jax import lax
from jax.experimental import pallas as pl
from jax.experimental.pallas import tpu as pltpu
from jax.experimental.pallas import tpu_sc as plsc

N = 50000
E = 800000
H = 64
Q = 16
CH = 80
NTILES = 16
CPT = E // CH // NTILES
NPAD = 50048
RPT = NPAD // NTILES
BLK = 1000


def _fill(buf, nrows, val):
    v = jnp.full((16,), val, jnp.float32)

    def body(r, carry):
        buf[r, pl.ds(0, 16)] = v
        return carry

    lax.fori_loop(0, nrows, body, 0)


def _zero_acc(t, acc, zbuf):
    pltpu.sync_copy(zbuf, acc.at[pl.ds(t * RPT, RPT)])


def _count_loop(t, acc, dstbuf, obuf, dst_r):
    base = t * CPT * CH

    def chunk(j, carry):
        pltpu.sync_copy(dst_r.at[pl.ds(base + j * CH, CH)], dstbuf.at[0])
        pltpu.sync_copy(obuf, acc.at[dstbuf.at[0]], add=True)
        return carry

    lax.fori_loop(0, CPT, chunk, 0)


def _scatter_loop(t, acc, idxbuf, dstbuf, rowbuf, src_r, dst_r, table):
    base = t * CPT * CH

    def chunk(j, carry):
        off = base + j * CH
        pltpu.sync_copy(src_r.at[pl.ds(off, CH)], idxbuf.at[0])
        pltpu.sync_copy(dst_r.at[pl.ds(off, CH)], dstbuf.at[0])
        pltpu.sync_copy(table.at[idxbuf.at[0]], rowbuf.at[0])
        pltpu.sync_copy(rowbuf.at[0], acc.at[dstbuf.at[0]], add=True)
        return carry

    lax.fori_loop(0, CPT, chunk, 0)


def _dump(t, acc, out_ref, lead=None):
    sl = pl.ds(t * RPT, RPT)
    dst = out_ref.at[lead, sl] if lead is not None else out_ref.at[sl]
    pltpu.sync_copy(acc.at[sl], dst)


def _seg_phases(c, t, acc, zbuf, idxbuf, dstbuf, rowbuf, src_r, dst_r,
                tables, out_ref):
    for p in range(2):
        _zero_acc(t, acc, zbuf)
        plsc.subcore_barrier()

        @pl.when(c == 0)
        def _():
            _scatter_loop(t, acc, idxbuf, dstbuf, rowbuf, src_r, dst_r,
                          tables[p])

        @pl.when(c == 1)
        def _():
            _scatter_loop(t, acc, idxbuf, dstbuf, rowbuf, src_r, dst_r,
                          tables[2 + p])

        plsc.subcore_barrier()

        @pl.when(c == 0)
        def _():
            _dump(t, acc, out_ref, lead=p)

        @pl.when(c == 1)
        def _():
            _dump(t, acc, out_ref, lead=2 + p)

        plsc.subcore_barrier()


_SC_SCRATCH = [
    pltpu.VMEM_SHARED((NPAD, Q), jnp.float32),
    pltpu.VMEM((RPT, Q), jnp.float32),
    pltpu.VMEM((CH, Q), jnp.float32),
    pltpu.VMEM((1, CH), jnp.int32),
    pltpu.VMEM((1, CH), jnp.int32),
    pltpu.VMEM((1, CH, Q), jnp.float32),
]


def _layer0_body(src_sc, dst_sc, src_cs, dst_cs,
                 ts0, ts1, ts2, ts3, tc0, tc1, tc2, tc3,
                 cnt_code, cnt_stay, s_code, s_stay,
                 acc, zbuf, obuf, idxbuf, dstbuf, rowbuf):
    c = lax.axis_index("c")
    t = lax.axis_index("s")
    _fill(zbuf, RPT, 0.0)
    _fill(obuf, CH, 1.0)

    _zero_acc(t, acc, zbuf)
    plsc.subcore_barrier()

    @pl.when(c == 0)
    def _():
        _count_loop(t, acc, dstbuf, obuf, dst_sc)

    @pl.when(c == 1)
    def _():
        _count_loop(t, acc, dstbuf, obuf, dst_cs)

    plsc.subcore_barrier()

    @pl.when(c == 0)
    def _():
        _dump(t, acc, cnt_code)

    @pl.when(c == 1)
    def _():
        _dump(t, acc, cnt_stay)

    plsc.subcore_barrier()

    _seg_phases(c, t, acc, zbuf, idxbuf, dstbuf, rowbuf, src_sc, dst_sc,
                (ts0, ts1, ts2, ts3), s_code)

    _seg_phases(c, t, acc, zbuf, idxbuf, dstbuf, rowbuf, src_cs, dst_cs,
                (tc0, tc1, tc2, tc3), s_stay)


def _layer1_body(src_cs, dst_cs, tc0, tc1, tc2, tc3, s_stay,
                 acc, zbuf, idxbuf, dstbuf, rowbuf):
    c = lax.axis_index("c")
    t = lax.axis_index("s")
    _fill(zbuf, RPT, 0.0)
    _seg_phases(c, t, acc, zbuf, idxbuf, dstbuf, rowbuf, src_cs, dst_cs,
                (tc0, tc1, tc2, tc3), s_stay)


def _sc_layer0(src_sc, dst_sc, src_cs, dst_cs, ts, tc):
    f = pl.kernel(
        _layer0_body,
        out_type=[
            jax.ShapeDtypeStruct((NPAD, Q), jnp.float32),
            jax.ShapeDtypeStruct((NPAD, Q), jnp.float32),
            jax.ShapeDtypeStruct((4, NPAD, Q), jnp.float32),
            jax.ShapeDtypeStruct((4, NPAD, Q), jnp.float32),
        ],
        mesh=plsc.VectorSubcoreMesh(core_axis_name="c", subcore_axis_name="s"),
        scratch_types=_SC_SCRATCH,
        compiler_params=pltpu.CompilerParams(use_tc_tiling_on_sc=False),
    )
    return f(src_sc, dst_sc, src_cs, dst_cs, *ts, *tc)


def _sc_layer1(src_cs, dst_cs, tc):
    f = pl.kernel(
        _layer1_body,
        out_type=jax.ShapeDtypeStruct((4, NPAD, Q), jnp.float32),
        mesh=plsc.VectorSubcoreMesh(core_axis_name="c", subcore_axis_name="s"),
        scratch_types=[_SC_SCRATCH[0], _SC_SCRATCH[1]] + _SC_SCRATCH[3:],
        compiler_params=pltpu.CompilerParams(use_tc_tiling_on_sc=False),
    )
    return f(src_cs, dst_cs, *tc)


def _proj_body(x_ref, w_ref, b_ref, o0, o1, o2, o3):
    y = jnp.dot(x_ref[...], w_ref[...], preferred_element_type=jnp.float32)
    y = jnp.maximum(y + b_ref[...], 0.0)
    for q, o in enumerate((o0, o1, o2, o3)):
        o[...] = y[:, q * Q:(q + 1) * Q]


def _proj(x, w, b):
    d_in = x.shape[1]
    return pl.pallas_call(
        _proj_body,
        grid=(N // BLK,),
        in_specs=[
            pl.BlockSpec((BLK, d_in), lambda i: (i, 0)),
            pl.BlockSpec((d_in, H), lambda i: (0, 0)),
            pl.BlockSpec((1, H), lambda i: (0, 0)),
        ],
        out_specs=[pl.BlockSpec((BLK, Q), lambda i: (i, 0))] * 4,
        out_shape=[jax.ShapeDtypeStruct((N, Q), jnp.float32)] * 4,
    )(x, w, b.reshape(1, H))


def _sage_update(s_ref, cnt_ref, h_refs, wl_ref, bl_ref, wr_ref, br_ref,
                 g_ref, be_ref):
    cnt = jnp.maximum(cnt_ref[:, 0:1], 1.0)
    wl = wl_ref[...]
    wr = wr_ref[...]
    out = bl_ref[...] + br_ref[...]
    for q in range(4):
        out += jnp.dot(s_ref[q] / cnt, wl[q * Q:(q + 1) * Q],
                       preferred_element_type=jnp.float32)
        out += jnp.dot(h_refs[q][...], wr[q * Q:(q + 1) * Q],
                       preferred_element_type=jnp.float32)
    nrm = jnp.sqrt(jnp.sum(out * out, axis=-1, keepdims=True))
    out = out / jnp.maximum(nrm, 1e-12)
    r = jnp.maximum(out, 0.0)
    m = jnp.mean(r, axis=-1, keepdims=True)
    v = jnp.mean((r - m) ** 2, axis=-1, keepdims=True)
    return (r - m) / jnp.sqrt(v + 1e-5) * g_ref[...] + be_ref[...]


def _update_body(s_ref, cnt_ref, h0, h1, h2, h3, wl_ref, bl_ref, wr_ref,
                 br_ref, g_ref, be_ref, o0, o1, o2, o3):
    h = _sage_update(s_ref, cnt_ref, (h0, h1, h2, h3), wl_ref, bl_ref,
                     wr_ref, br_ref, g_ref, be_ref)
    for q, o in enumerate((o0, o1, o2, o3)):
        o[...] = h[:, q * Q:(q + 1) * Q]


def _final_body(s_ref, cnt_ref, h0, h1, h2, h3, wl_ref, bl_ref, wr_ref,
                br_ref, g_ref, be_ref, wc_ref, bc_ref, logits_ref):
    h = _sage_update(s_ref, cnt_ref, (h0, h1, h2, h3), wl_ref, bl_ref,
                     wr_ref, br_ref, g_ref, be_ref)
    logits_ref[...] = (jnp.dot(h, wc_ref[...], preferred_element_type=jnp.float32)
                       + bc_ref[...])


_W_SPECS = [
    pl.BlockSpec((H, H), lambda i: (0, 0)),
    pl.BlockSpec((1, H), lambda i: (0, 0)),
    pl.BlockSpec((H, H), lambda i: (0, 0)),
    pl.BlockSpec((1, H), lambda i: (0, 0)),
    pl.BlockSpec((1, H), lambda i: (0, 0)),
    pl.BlockSpec((1, H), lambda i: (0, 0)),
]

_IN_SPECS = [
    pl.BlockSpec((4, BLK, Q), lambda i: (0, i, 0)),
    pl.BlockSpec((BLK, Q), lambda i: (i, 0)),
    pl.BlockSpec((BLK, Q), lambda i: (i, 0)),
    pl.BlockSpec((BLK, Q), lambda i: (i, 0)),
    pl.BlockSpec((BLK, Q), lambda i: (i, 0)),
    pl.BlockSpec((BLK, Q), lambda i: (i, 0)),
]


def _update(s, cnt, h, wl, bl, wr, br, g, be):
    return pl.pallas_call(
        _update_body,
        grid=(N // BLK,),
        in_specs=_IN_SPECS + _W_SPECS,
        out_specs=[pl.BlockSpec((BLK, Q), lambda i: (i, 0))] * 4,
        out_shape=[jax.ShapeDtypeStruct((N, Q), jnp.float32)] * 4,
    )(s, cnt, *h, wl, bl.reshape(1, -1), wr, br.reshape(1, -1),
      g.reshape(1, -1), be.reshape(1, -1))


def _final(s, cnt, h, wl, bl, wr, br, g, be, wc, bc):
    n_cls = wc.shape[1]
    return pl.pallas_call(
        _final_body,
        grid=(N // BLK,),
        in_specs=_IN_SPECS + _W_SPECS + [
            pl.BlockSpec((H, n_cls), lambda i: (0, 0)),
            pl.BlockSpec((1, n_cls), lambda i: (0, 0)),
        ],
        out_specs=pl.BlockSpec((BLK, n_cls), lambda i: (i, 0)),
        out_shape=jax.ShapeDtypeStruct((N, n_cls), jnp.float32),
    )(s, cnt, *h, wl, bl.reshape(1, -1), wr, br.reshape(1, -1),
      g.reshape(1, -1), be.reshape(1, -1), wc, bc.reshape(1, -1))


def kernel(x_stay, x_code, ei_sc, ei_cs, W_in_stay, b_in_stay, W_in_code,
           b_in_code, Wl0_sc, bl0_sc, Wr0_sc, br0_sc, Wl0_cs, bl0_cs, Wr0_cs,
           br0_cs, g0, be0, Wl1_sc, bl1_sc, Wr1_sc, br1_sc, Wl1_cs, bl1_cs,
           Wr1_cs, br1_cs, g1, be1, Wc, bc):
    src_sc, dst_sc = ei_sc[0], ei_sc[1]
    src_cs, dst_cs = ei_cs[0], ei_cs[1]

    hs = _proj(x_stay, W_in_stay, b_in_stay)
    hc = _proj(x_code, W_in_code, b_in_code)

    cnt_code, cnt_stay, s_code, s_stay = _sc_layer0(
        src_sc, dst_sc, src_cs, dst_cs, hs, hc)

    hc1 = _update(s_code, cnt_code, hc, Wl0_sc, bl0_sc, Wr0_sc, br0_sc,
                  g0, be0)
    hs1 = _update(s_stay, cnt_stay, hs, Wl0_cs, bl0_cs, Wr0_cs, br0_cs,
                  g0, be0)

    s_stay1 = _sc_layer1(src_cs, dst_cs, hc1)

    return _final(s_stay1, cnt_stay, hs1, Wl1_cs, bl1_cs, Wr1_cs, br1_cs,
                  g1, be1, Wc, bc)


# speedup vs baseline: 6.4472x; 4.4197x over previous
import jax
import jax.numpy as jnp
from jax import lax
from jax.experimental import pallas as pl
from jax.experimental.pallas import tpu as pltpu
from jax.experimental.pallas import tpu_sc as plsc

N = 50000
E = 800000
H = 64
Q = 16
CH = 80
NTILES = 16
CPT = E // CH // NTILES
GRP = 25
NG = CPT // GRP
NPAD = 50048
RPT = NPAD // NTILES
ZB = RPT // 8
BLK = 1000


def _fill(buf, nrows, val):
    v = jnp.full((16,), val, jnp.float32)

    def body(r, carry):
        buf[r, pl.ds(0, 16)] = v
        return carry

    lax.fori_loop(0, nrows, body, 0)


def _zero_acc(t, acc, zbuf):
    for k in range(RPT // ZB):
        pltpu.sync_copy(zbuf, acc.at[pl.ds(t * RPT + k * ZB, ZB)])


def _count_loop(t, acc, dstblk, obuf, dst_r, ssem):
    base = t * CPT

    def group(g, carry):
        pltpu.sync_copy(dst_r.at[pl.ds(base + g * GRP, GRP)], dstblk)

        def fire(i, cc):
            pltpu.async_copy(obuf, acc.at[dstblk.at[i]], ssem, add=True)
            return cc

        lax.fori_loop(0, GRP, fire, 0)

        def drain(i, cc):
            pltpu.make_async_copy(obuf, acc.at[dstblk.at[i]], ssem).wait()
            return cc

        lax.fori_loop(0, GRP, drain, 0)
        return carry

    lax.fori_loop(0, NG, group, 0)


def _scatter_loop(t, acc, idxblk, dstblk, rowbuf, src_r, dst_r, table,
                  gsem, ssem):
    base = t * CPT

    def group(g, carry):
        gb = base + g * GRP
        pltpu.sync_copy(src_r.at[pl.ds(gb, GRP)], idxblk)
        pltpu.sync_copy(dst_r.at[pl.ds(gb, GRP)], dstblk)

        def fire(i, cc):
            pltpu.async_copy(table.at[idxblk.at[i]], rowbuf.at[i], gsem)
            return cc

        lax.fori_loop(0, GRP, fire, 0)

        def wait_fire(i, cc):
            pltpu.make_async_copy(table.at[idxblk.at[i]], rowbuf.at[i],
                                  gsem).wait()
            pltpu.async_copy(rowbuf.at[i], acc.at[dstblk.at[i]], ssem,
                             add=True)
            return cc

        lax.fori_loop(0, GRP, wait_fire, 0)

        def drain(i, cc):
            pltpu.make_async_copy(rowbuf.at[i], acc.at[dstblk.at[i]],
                                  ssem).wait()
            return cc

        lax.fori_loop(0, GRP, drain, 0)
        return carry

    lax.fori_loop(0, NG, group, 0)


def _dump(t, acc, out_ref, lead=None):
    sl = pl.ds(t * RPT, RPT)
    dst = out_ref.at[lead, sl] if lead is not None else out_ref.at[sl]
    pltpu.sync_copy(acc.at[sl], dst)


def _seg_phases(c, t, acc, zbuf, idxblk, dstblk, rowbuf, gsem, ssem,
                src_r, dst_r, tables, out_ref):
    for p in range(2):
        _zero_acc(t, acc, zbuf)
        plsc.subcore_barrier()

        @pl.when(c == 0)
        def _():
            _scatter_loop(t, acc, idxblk, dstblk, rowbuf, src_r, dst_r,
                          tables[p], gsem, ssem)

        @pl.when(c == 1)
        def _():
            _scatter_loop(t, acc, idxblk, dstblk, rowbuf, src_r, dst_r,
                          tables[2 + p], gsem, ssem)

        plsc.subcore_barrier()

        @pl.when(c == 0)
        def _():
            _dump(t, acc, out_ref, lead=p)

        @pl.when(c == 1)
        def _():
            _dump(t, acc, out_ref, lead=2 + p)

        plsc.subcore_barrier()


_SC_SCRATCH = [
    pltpu.VMEM_SHARED((NPAD, Q), jnp.float32),
    pltpu.VMEM((ZB, Q), jnp.float32),
    pltpu.VMEM((CH, Q), jnp.float32),
    pltpu.VMEM((GRP, CH), jnp.int32),
    pltpu.VMEM((GRP, CH), jnp.int32),
    pltpu.VMEM((GRP, CH, Q), jnp.float32),
    pltpu.SemaphoreType.DMA,
    pltpu.SemaphoreType.DMA,
]


def _layer0_body(src_sc, dst_sc, src_cs, dst_cs,
                 ts0, ts1, ts2, ts3, tc0, tc1, tc2, tc3,
                 cnt_code, cnt_stay, s_code, s_stay,
                 acc, zbuf, obuf, idxblk, dstblk, rowbuf, gsem, ssem):
    c = lax.axis_index("c")
    t = lax.axis_index("s")
    _fill(zbuf, ZB, 0.0)
    _fill(obuf, CH, 1.0)

    _zero_acc(t, acc, zbuf)
    plsc.subcore_barrier()

    @pl.when(c == 0)
    def _():
        _count_loop(t, acc, dstblk, obuf, dst_sc, ssem)

    @pl.when(c == 1)
    def _():
        _count_loop(t, acc, dstblk, obuf, dst_cs, ssem)

    plsc.subcore_barrier()

    @pl.when(c == 0)
    def _():
        _dump(t, acc, cnt_code)

    @pl.when(c == 1)
    def _():
        _dump(t, acc, cnt_stay)

    plsc.subcore_barrier()

    _seg_phases(c, t, acc, zbuf, idxblk, dstblk, rowbuf, gsem, ssem,
                src_sc, dst_sc, (ts0, ts1, ts2, ts3), s_code)

    _seg_phases(c, t, acc, zbuf, idxblk, dstblk, rowbuf, gsem, ssem,
                src_cs, dst_cs, (tc0, tc1, tc2, tc3), s_stay)


def _layer1_body(src_cs, dst_cs, tc0, tc1, tc2, tc3, s_stay,
                 acc, zbuf, idxblk, dstblk, rowbuf, gsem, ssem):
    c = lax.axis_index("c")
    t = lax.axis_index("s")
    _fill(zbuf, ZB, 0.0)
    _seg_phases(c, t, acc, zbuf, idxblk, dstblk, rowbuf, gsem, ssem,
                src_cs, dst_cs, (tc0, tc1, tc2, tc3), s_stay)


def _sc_layer0(src_sc, dst_sc, src_cs, dst_cs, ts, tc):
    f = pl.kernel(
        _layer0_body,
        out_type=[
            jax.ShapeDtypeStruct((NPAD, Q), jnp.float32),
            jax.ShapeDtypeStruct((NPAD, Q), jnp.float32),
            jax.ShapeDtypeStruct((4, NPAD, Q), jnp.float32),
            jax.ShapeDtypeStruct((4, NPAD, Q), jnp.float32),
        ],
        mesh=plsc.VectorSubcoreMesh(core_axis_name="c", subcore_axis_name="s"),
        scratch_types=_SC_SCRATCH,
        compiler_params=pltpu.CompilerParams(use_tc_tiling_on_sc=False),
    )
    return f(src_sc, dst_sc, src_cs, dst_cs, *ts, *tc)


def _sc_layer1(src_cs, dst_cs, tc):
    f = pl.kernel(
        _layer1_body,
        out_type=jax.ShapeDtypeStruct((4, NPAD, Q), jnp.float32),
        mesh=plsc.VectorSubcoreMesh(core_axis_name="c", subcore_axis_name="s"),
        scratch_types=[_SC_SCRATCH[0], _SC_SCRATCH[1]] + _SC_SCRATCH[3:],
        compiler_params=pltpu.CompilerParams(use_tc_tiling_on_sc=False),
    )
    return f(src_cs, dst_cs, *tc)


def _proj_body(x_ref, w_ref, b_ref, o0, o1, o2, o3):
    y = jnp.dot(x_ref[...], w_ref[...], preferred_element_type=jnp.float32)
    y = jnp.maximum(y + b_ref[...], 0.0)
    for q, o in enumerate((o0, o1, o2, o3)):
        o[...] = y[:, q * Q:(q + 1) * Q]


def _proj(x, w, b):
    d_in = x.shape[1]
    return pl.pallas_call(
        _proj_body,
        grid=(N // BLK,),
        in_specs=[
            pl.BlockSpec((BLK, d_in), lambda i: (i, 0)),
            pl.BlockSpec((d_in, H), lambda i: (0, 0)),
            pl.BlockSpec((1, H), lambda i: (0, 0)),
        ],
        out_specs=[pl.BlockSpec((BLK, Q), lambda i: (i, 0))] * 4,
        out_shape=[jax.ShapeDtypeStruct((N, Q), jnp.float32)] * 4,
    )(x, w, b.reshape(1, H))


def _sage_update(s_ref, cnt_ref, h_refs, wl_ref, bl_ref, wr_ref, br_ref,
                 g_ref, be_ref):
    cnt = jnp.maximum(cnt_ref[:, 0:1], 1.0)
    wl = wl_ref[...]
    wr = wr_ref[...]
    out = bl_ref[...] + br_ref[...]
    for q in range(4):
        out += jnp.dot(s_ref[q] / cnt, wl[q * Q:(q + 1) * Q],
                       preferred_element_type=jnp.float32)
        out += jnp.dot(h_refs[q][...], wr[q * Q:(q + 1) * Q],
                       preferred_element_type=jnp.float32)
    nrm = jnp.sqrt(jnp.sum(out * out, axis=-1, keepdims=True))
    out = out / jnp.maximum(nrm, 1e-12)
    r = jnp.maximum(out, 0.0)
    m = jnp.mean(r, axis=-1, keepdims=True)
    v = jnp.mean((r - m) ** 2, axis=-1, keepdims=True)
    return (r - m) / jnp.sqrt(v + 1e-5) * g_ref[...] + be_ref[...]


def _update_body(s_ref, cnt_ref, h0, h1, h2, h3, wl_ref, bl_ref, wr_ref,
                 br_ref, g_ref, be_ref, o0, o1, o2, o3):
    h = _sage_update(s_ref, cnt_ref, (h0, h1, h2, h3), wl_ref, bl_ref,
                     wr_ref, br_ref, g_ref, be_ref)
    for q, o in enumerate((o0, o1, o2, o3)):
        o[...] = h[:, q * Q:(q + 1) * Q]


def _final_body(s_ref, cnt_ref, h0, h1, h2, h3, wl_ref, bl_ref, wr_ref,
                br_ref, g_ref, be_ref, wc_ref, bc_ref, logits_ref):
    h = _sage_update(s_ref, cnt_ref, (h0, h1, h2, h3), wl_ref, bl_ref,
                     wr_ref, br_ref, g_ref, be_ref)
    logits_ref[...] = (jnp.dot(h, wc_ref[...], preferred_element_type=jnp.float32)
                       + bc_ref[...])


_W_SPECS = [
    pl.BlockSpec((H, H), lambda i: (0, 0)),
    pl.BlockSpec((1, H), lambda i: (0, 0)),
    pl.BlockSpec((H, H), lambda i: (0, 0)),
    pl.BlockSpec((1, H), lambda i: (0, 0)),
    pl.BlockSpec((1, H), lambda i: (0, 0)),
    pl.BlockSpec((1, H), lambda i: (0, 0)),
]

_IN_SPECS = [
    pl.BlockSpec((4, BLK, Q), lambda i: (0, i, 0)),
    pl.BlockSpec((BLK, Q), lambda i: (i, 0)),
    pl.BlockSpec((BLK, Q), lambda i: (i, 0)),
    pl.BlockSpec((BLK, Q), lambda i: (i, 0)),
    pl.BlockSpec((BLK, Q), lambda i: (i, 0)),
    pl.BlockSpec((BLK, Q), lambda i: (i, 0)),
]


def _update(s, cnt, h, wl, bl, wr, br, g, be):
    return pl.pallas_call(
        _update_body,
        grid=(N // BLK,),
        in_specs=_IN_SPECS + _W_SPECS,
        out_specs=[pl.BlockSpec((BLK, Q), lambda i: (i, 0))] * 4,
        out_shape=[jax.ShapeDtypeStruct((N, Q), jnp.float32)] * 4,
    )(s, cnt, *h, wl, bl.reshape(1, -1), wr, br.reshape(1, -1),
      g.reshape(1, -1), be.reshape(1, -1))


def _final(s, cnt, h, wl, bl, wr, br, g, be, wc, bc):
    n_cls = wc.shape[1]
    return pl.pallas_call(
        _final_body,
        grid=(N // BLK,),
        in_specs=_IN_SPECS + _W_SPECS + [
            pl.BlockSpec((H, n_cls), lambda i: (0, 0)),
            pl.BlockSpec((1, n_cls), lambda i: (0, 0)),
        ],
        out_specs=pl.BlockSpec((BLK, n_cls), lambda i: (i, 0)),
        out_shape=jax.ShapeDtypeStruct((N, n_cls), jnp.float32),
    )(s, cnt, *h, wl, bl.reshape(1, -1), wr, br.reshape(1, -1),
      g.reshape(1, -1), be.reshape(1, -1), wc, bc.reshape(1, -1))


def kernel(x_stay, x_code, ei_sc, ei_cs, W_in_stay, b_in_stay, W_in_code,
           b_in_code, Wl0_sc, bl0_sc, Wr0_sc, br0_sc, Wl0_cs, bl0_cs, Wr0_cs,
           br0_cs, g0, be0, Wl1_sc, bl1_sc, Wr1_sc, br1_sc, Wl1_cs, bl1_cs,
           Wr1_cs, br1_cs, g1, be1, Wc, bc):
    src_sc = ei_sc[0].reshape(E // CH, CH)
    dst_sc = ei_sc[1].reshape(E // CH, CH)
    src_cs = ei_cs[0].reshape(E // CH, CH)
    dst_cs = ei_cs[1].reshape(E // CH, CH)

    hs = _proj(x_stay, W_in_stay, b_in_stay)
    hc = _proj(x_code, W_in_code, b_in_code)

    cnt_code, cnt_stay, s_code, s_stay = _sc_layer0(
        src_sc, dst_sc, src_cs, dst_cs, hs, hc)

    hc1 = _update(s_code, cnt_code, hc, Wl0_sc, bl0_sc, Wr0_sc, br0_sc,
                  g0, be0)
    hs1 = _update(s_stay, cnt_stay, hs, Wl0_cs, bl0_cs, Wr0_cs, br0_cs,
                  g0, be0)

    s_stay1 = _sc_layer1(src_cs, dst_cs, hc1)

    return _final(s_stay1, cnt_stay, hs1, Wl1_cs, bl1_cs, Wr1_cs, br1_cs,
                  g1, be1, Wc, bc)


# speedup vs baseline: 6.4509x; 1.0006x over previous
import jax
import jax.numpy as jnp
from jax import lax
from jax.experimental import pallas as pl
from jax.experimental.pallas import tpu as pltpu
from jax.experimental.pallas import tpu_sc as plsc

N = 50000
E = 800000
H = 64
Q = 16
CH = 80
NTILES = 16
CPT = E // CH // NTILES
GRP = 25
NG = CPT // GRP
NPAD = 50048
RPT = NPAD // NTILES
ZB = RPT // 8
BLK = 1000


def _fill(buf, nrows, val):
    v = jnp.full((16,), val, jnp.float32)

    def body(r, carry):
        buf[r, pl.ds(0, 16)] = v
        return carry

    lax.fori_loop(0, nrows, body, 0)


def _zero_acc(t, acc, zbuf):
    for k in range(RPT // ZB):
        pltpu.sync_copy(zbuf, acc.at[pl.ds(t * RPT + k * ZB, ZB)])


def _count_loop(t, acc, dstblk, obuf, dst_r, ssem):
    base = t * CPT

    def group(g, carry):
        pltpu.sync_copy(dst_r.at[pl.ds(base + g * GRP, GRP)], dstblk)

        def fire(i, cc):
            pltpu.async_copy(obuf, acc.at[dstblk.at[i]], ssem, add=True)
            return cc

        lax.fori_loop(0, GRP, fire, 0)

        def drain(i, cc):
            pltpu.make_async_copy(obuf, acc.at[dstblk.at[i]], ssem).wait()
            return cc

        lax.fori_loop(0, GRP, drain, 0)
        return carry

    lax.fori_loop(0, NG, group, 0)


def _scatter_loop(t, acc, idxblk, dstblk, rowbuf, src_r, dst_r, table,
                  gsem, ssem):
    base = t * CPT

    def group(g, carry):
        gb = base + g * GRP
        pltpu.sync_copy(src_r.at[pl.ds(gb, GRP)], idxblk)
        pltpu.sync_copy(dst_r.at[pl.ds(gb, GRP)], dstblk)

        def fire(i, cc):
            pltpu.async_copy(table.at[idxblk.at[i]], rowbuf.at[i], gsem)
            return cc

        lax.fori_loop(0, GRP, fire, 0)

        def wait_fire(i, cc):
            pltpu.make_async_copy(table.at[idxblk.at[i]], rowbuf.at[i],
                                  gsem).wait()
            pltpu.async_copy(rowbuf.at[i], acc.at[dstblk.at[i]], ssem,
                             add=True)
            return cc

        lax.fori_loop(0, GRP, wait_fire, 0)

        def drain(i, cc):
            pltpu.make_async_copy(rowbuf.at[i], acc.at[dstblk.at[i]],
                                  ssem).wait()
            return cc

        lax.fori_loop(0, GRP, drain, 0)
        return carry

    lax.fori_loop(0, NG, group, 0)


def _dump(t, acc, out_ref, lead=None):
    sl = pl.ds(t * RPT, RPT)
    dst = out_ref.at[lead, sl] if lead is not None else out_ref.at[sl]
    pltpu.sync_copy(acc.at[sl], dst)


def _seg_phases(c, t, acc, zbuf, idxblk, dstblk, rowbuf, gsem, ssem,
                src_r, dst_r, tables, out_ref):
    for p in range(2):
        _zero_acc(t, acc, zbuf)
        plsc.subcore_barrier()

        @pl.when(c == 0)
        def _():
            _scatter_loop(t, acc, idxblk, dstblk, rowbuf, src_r, dst_r,
                          tables[p], gsem, ssem)

        @pl.when(c == 1)
        def _():
            _scatter_loop(t, acc, idxblk, dstblk, rowbuf, src_r, dst_r,
                          tables[2 + p], gsem, ssem)

        plsc.subcore_barrier()

        @pl.when(c == 0)
        def _():
            _dump(t, acc, out_ref, lead=p)

        @pl.when(c == 1)
        def _():
            _dump(t, acc, out_ref, lead=2 + p)

        plsc.subcore_barrier()


_SC_SCRATCH = [
    pltpu.VMEM_SHARED((NPAD, Q), jnp.float32),
    pltpu.VMEM((ZB, Q), jnp.float32),
    pltpu.VMEM((CH, Q), jnp.float32),
    pltpu.VMEM((GRP, CH), jnp.int32),
    pltpu.VMEM((GRP, CH), jnp.int32),
    pltpu.VMEM((GRP, CH, Q), jnp.float32),
]


def _layer0_body(src_sc, dst_sc, src_cs, dst_cs,
                 ts0, ts1, ts2, ts3, tc0, tc1, tc2, tc3,
                 cnt_code, cnt_stay, s_code, s_stay,
                 acc, zbuf, obuf, idxblk, dstblk, rowbuf):
    def scoped(gsem, ssem):
        _layer0_inner(src_sc, dst_sc, src_cs, dst_cs,
                      ts0, ts1, ts2, ts3, tc0, tc1, tc2, tc3,
                      cnt_code, cnt_stay, s_code, s_stay,
                      acc, zbuf, obuf, idxblk, dstblk, rowbuf, gsem, ssem)

    pl.run_scoped(scoped, pltpu.SemaphoreType.DMA, pltpu.SemaphoreType.DMA)


def _layer0_inner(src_sc, dst_sc, src_cs, dst_cs,
                  ts0, ts1, ts2, ts3, tc0, tc1, tc2, tc3,
                  cnt_code, cnt_stay, s_code, s_stay,
                  acc, zbuf, obuf, idxblk, dstblk, rowbuf, gsem, ssem):
    c = lax.axis_index("c")
    t = lax.axis_index("s")
    _fill(zbuf, ZB, 0.0)
    _fill(obuf, CH, 1.0)

    _zero_acc(t, acc, zbuf)
    plsc.subcore_barrier()

    @pl.when(c == 0)
    def _():
        _count_loop(t, acc, dstblk, obuf, dst_sc, ssem)

    @pl.when(c == 1)
    def _():
        _count_loop(t, acc, dstblk, obuf, dst_cs, ssem)

    plsc.subcore_barrier()

    @pl.when(c == 0)
    def _():
        _dump(t, acc, cnt_code)

    @pl.when(c == 1)
    def _():
        _dump(t, acc, cnt_stay)

    plsc.subcore_barrier()

    _seg_phases(c, t, acc, zbuf, idxblk, dstblk, rowbuf, gsem, ssem,
                src_sc, dst_sc, (ts0, ts1, ts2, ts3), s_code)

    _seg_phases(c, t, acc, zbuf, idxblk, dstblk, rowbuf, gsem, ssem,
                src_cs, dst_cs, (tc0, tc1, tc2, tc3), s_stay)


def _layer1_body(src_cs, dst_cs, tc0, tc1, tc2, tc3, s_stay,
                 acc, zbuf, idxblk, dstblk, rowbuf):
    def scoped(gsem, ssem):
        c = lax.axis_index("c")
        t = lax.axis_index("s")
        _fill(zbuf, ZB, 0.0)
        _seg_phases(c, t, acc, zbuf, idxblk, dstblk, rowbuf, gsem, ssem,
                    src_cs, dst_cs, (tc0, tc1, tc2, tc3), s_stay)

    pl.run_scoped(scoped, pltpu.SemaphoreType.DMA, pltpu.SemaphoreType.DMA)


def _sc_layer0(src_sc, dst_sc, src_cs, dst_cs, ts, tc):
    f = pl.kernel(
        _layer0_body,
        out_type=[
            jax.ShapeDtypeStruct((NPAD, Q), jnp.float32),
            jax.ShapeDtypeStruct((NPAD, Q), jnp.float32),
            jax.ShapeDtypeStruct((4, NPAD, Q), jnp.float32),
            jax.ShapeDtypeStruct((4, NPAD, Q), jnp.float32),
        ],
        mesh=plsc.VectorSubcoreMesh(core_axis_name="c", subcore_axis_name="s"),
        scratch_types=_SC_SCRATCH,
        compiler_params=pltpu.CompilerParams(use_tc_tiling_on_sc=False),
    )
    return f(src_sc, dst_sc, src_cs, dst_cs, *ts, *tc)


def _sc_layer1(src_cs, dst_cs, tc):
    f = pl.kernel(
        _layer1_body,
        out_type=jax.ShapeDtypeStruct((4, NPAD, Q), jnp.float32),
        mesh=plsc.VectorSubcoreMesh(core_axis_name="c", subcore_axis_name="s"),
        scratch_types=[_SC_SCRATCH[0], _SC_SCRATCH[1]] + _SC_SCRATCH[3:],
        compiler_params=pltpu.CompilerParams(use_tc_tiling_on_sc=False),
    )
    return f(src_cs, dst_cs, *tc)


def _proj_body(x_ref, w_ref, b_ref, o0, o1, o2, o3):
    y = jnp.dot(x_ref[...], w_ref[...], preferred_element_type=jnp.float32)
    y = jnp.maximum(y + b_ref[...], 0.0)
    for q, o in enumerate((o0, o1, o2, o3)):
        o[...] = y[:, q * Q:(q + 1) * Q]


def _proj(x, w, b):
    d_in = x.shape[1]
    return pl.pallas_call(
        _proj_body,
        grid=(N // BLK,),
        in_specs=[
            pl.BlockSpec((BLK, d_in), lambda i: (i, 0)),
            pl.BlockSpec((d_in, H), lambda i: (0, 0)),
            pl.BlockSpec((1, H), lambda i: (0, 0)),
        ],
        out_specs=[pl.BlockSpec((BLK, Q), lambda i: (i, 0))] * 4,
        out_shape=[jax.ShapeDtypeStruct((N, Q), jnp.float32)] * 4,
    )(x, w, b.reshape(1, H))


def _sage_update(s_ref, cnt_ref, h_refs, wl_ref, bl_ref, wr_ref, br_ref,
                 g_ref, be_ref):
    cnt = jnp.maximum(cnt_ref[:, 0:1], 1.0)
    wl = wl_ref[...]
    wr = wr_ref[...]
    out = bl_ref[...] + br_ref[...]
    for q in range(4):
        out += jnp.dot(s_ref[q] / cnt, wl[q * Q:(q + 1) * Q],
                       preferred_element_type=jnp.float32)
        out += jnp.dot(h_refs[q][...], wr[q * Q:(q + 1) * Q],
                       preferred_element_type=jnp.float32)
    nrm = jnp.sqrt(jnp.sum(out * out, axis=-1, keepdims=True))
    out = out / jnp.maximum(nrm, 1e-12)
    r = jnp.maximum(out, 0.0)
    m = jnp.mean(r, axis=-1, keepdims=True)
    v = jnp.mean((r - m) ** 2, axis=-1, keepdims=True)
    return (r - m) / jnp.sqrt(v + 1e-5) * g_ref[...] + be_ref[...]


def _update_body(s_ref, cnt_ref, h0, h1, h2, h3, wl_ref, bl_ref, wr_ref,
                 br_ref, g_ref, be_ref, o0, o1, o2, o3):
    h = _sage_update(s_ref, cnt_ref, (h0, h1, h2, h3), wl_ref, bl_ref,
                     wr_ref, br_ref, g_ref, be_ref)
    for q, o in enumerate((o0, o1, o2, o3)):
        o[...] = h[:, q * Q:(q + 1) * Q]


def _final_body(s_ref, cnt_ref, h0, h1, h2, h3, wl_ref, bl_ref, wr_ref,
                br_ref, g_ref, be_ref, wc_ref, bc_ref, logits_ref):
    h = _sage_update(s_ref, cnt_ref, (h0, h1, h2, h3), wl_ref, bl_ref,
                     wr_ref, br_ref, g_ref, be_ref)
    logits_ref[...] = (jnp.dot(h, wc_ref[...], preferred_element_type=jnp.float32)
                       + bc_ref[...])


_W_SPECS = [
    pl.BlockSpec((H, H), lambda i: (0, 0)),
    pl.BlockSpec((1, H), lambda i: (0, 0)),
    pl.BlockSpec((H, H), lambda i: (0, 0)),
    pl.BlockSpec((1, H), lambda i: (0, 0)),
    pl.BlockSpec((1, H), lambda i: (0, 0)),
    pl.BlockSpec((1, H), lambda i: (0, 0)),
]

_IN_SPECS = [
    pl.BlockSpec((4, BLK, Q), lambda i: (0, i, 0)),
    pl.BlockSpec((BLK, Q), lambda i: (i, 0)),
    pl.BlockSpec((BLK, Q), lambda i: (i, 0)),
    pl.BlockSpec((BLK, Q), lambda i: (i, 0)),
    pl.BlockSpec((BLK, Q), lambda i: (i, 0)),
    pl.BlockSpec((BLK, Q), lambda i: (i, 0)),
]


def _update(s, cnt, h, wl, bl, wr, br, g, be):
    return pl.pallas_call(
        _update_body,
        grid=(N // BLK,),
        in_specs=_IN_SPECS + _W_SPECS,
        out_specs=[pl.BlockSpec((BLK, Q), lambda i: (i, 0))] * 4,
        out_shape=[jax.ShapeDtypeStruct((N, Q), jnp.float32)] * 4,
    )(s, cnt, *h, wl, bl.reshape(1, -1), wr, br.reshape(1, -1),
      g.reshape(1, -1), be.reshape(1, -1))


def _final(s, cnt, h, wl, bl, wr, br, g, be, wc, bc):
    n_cls = wc.shape[1]
    return pl.pallas_call(
        _final_body,
        grid=(N // BLK,),
        in_specs=_IN_SPECS + _W_SPECS + [
            pl.BlockSpec((H, n_cls), lambda i: (0, 0)),
            pl.BlockSpec((1, n_cls), lambda i: (0, 0)),
        ],
        out_specs=pl.BlockSpec((BLK, n_cls), lambda i: (i, 0)),
        out_shape=jax.ShapeDtypeStruct((N, n_cls), jnp.float32),
    )(s, cnt, *h, wl, bl.reshape(1, -1), wr, br.reshape(1, -1),
      g.reshape(1, -1), be.reshape(1, -1), wc, bc.reshape(1, -1))


def kernel(x_stay, x_code, ei_sc, ei_cs, W_in_stay, b_in_stay, W_in_code,
           b_in_code, Wl0_sc, bl0_sc, Wr0_sc, br0_sc, Wl0_cs, bl0_cs, Wr0_cs,
           br0_cs, g0, be0, Wl1_sc, bl1_sc, Wr1_sc, br1_sc, Wl1_cs, bl1_cs,
           Wr1_cs, br1_cs, g1, be1, Wc, bc):
    src_sc = ei_sc[0].reshape(E // CH, CH)
    dst_sc = ei_sc[1].reshape(E // CH, CH)
    src_cs = ei_cs[0].reshape(E // CH, CH)
    dst_cs = ei_cs[1].reshape(E // CH, CH)

    hs = _proj(x_stay, W_in_stay, b_in_stay)
    hc = _proj(x_code, W_in_code, b_in_code)

    cnt_code, cnt_stay, s_code, s_stay = _sc_layer0(
        src_sc, dst_sc, src_cs, dst_cs, hs, hc)

    hc1 = _update(s_code, cnt_code, hc, Wl0_sc, bl0_sc, Wr0_sc, br0_sc,
                  g0, be0)
    hs1 = _update(s_stay, cnt_stay, hs, Wl0_cs, bl0_cs, Wr0_cs, br0_cs,
                  g0, be0)

    s_stay1 = _sc_layer1(src_cs, dst_cs, hc1)

    return _final(s_stay1, cnt_stay, hs1, Wl1_cs, bl1_cs, Wr1_cs, br1_cs,
                  g1, be1, Wc, bc)


# speedup vs baseline: 7.1993x; 1.1160x over previous
import jax
import jax.numpy as jnp
from jax import lax
from jax.experimental import pallas as pl
from jax.experimental.pallas import tpu as pltpu
from jax.experimental.pallas import tpu_sc as plsc

N = 50000
E = 800000
H = 64
Q = 16
CH = 80
NTILES = 16
CPT = E // CH // NTILES
GRP = 25
NG = CPT // GRP
NPAD = 50048
RPT = NPAD // NTILES
ZB = RPT // 8
BLK = 1000


def _fill(buf, nrows, val):
    v = jnp.full((16,), val, jnp.float32)

    def body(r, carry):
        buf[r, pl.ds(0, 16)] = v
        return carry

    lax.fori_loop(0, nrows, body, 0)


def _zero_acc(t, acc, zbuf):
    for k in range(RPT // ZB):
        pltpu.sync_copy(zbuf, acc.at[pl.ds(t * RPT + k * ZB, ZB)])


def _count_loop(t, acc, dstblk, obuf, dst_r, ssem):
    base = t * CPT

    def group(g, carry):
        pltpu.sync_copy(dst_r.at[pl.ds(base + g * GRP, GRP)], dstblk.at[0])

        def fire(i, cc):
            pltpu.async_copy(obuf, acc.at[dstblk.at[0, i]], ssem, add=True)
            return cc

        lax.fori_loop(0, GRP, fire, 0)

        def drain(i, cc):
            pltpu.make_async_copy(obuf, acc.at[dstblk.at[0, i]],
                                  ssem).wait()
            return cc

        lax.fori_loop(0, GRP, drain, 0)
        return carry

    lax.fori_loop(0, NG, group, 0)


def _scatter_pipeline(t, acc, idxblk, dstblk, rowbuf, src_r, dst_r, table,
                      ga, gb, sa, sb, isem):
    base = t * CPT

    def fire_idx(g, b):
        sl = pl.ds(base + g * GRP, GRP)
        pltpu.async_copy(src_r.at[sl], idxblk.at[b], isem)
        pltpu.async_copy(dst_r.at[sl], dstblk.at[b], isem)

    def wait_idx(g, b):
        sl = pl.ds(base + g * GRP, GRP)
        pltpu.make_async_copy(src_r.at[sl], idxblk.at[b], isem).wait()
        pltpu.make_async_copy(dst_r.at[sl], dstblk.at[b], isem).wait()

    def fire_gathers(b, gsem):
        def fire(i, cc):
            pltpu.async_copy(table.at[idxblk.at[b, i]], rowbuf.at[b, i],
                             gsem)
            return cc

        lax.fori_loop(0, GRP, fire, 0)

    def wait_fire_scatters(b, gsem, ssem):
        def wf(i, cc):
            pltpu.make_async_copy(table.at[idxblk.at[b, i]],
                                  rowbuf.at[b, i], gsem).wait()
            pltpu.async_copy(rowbuf.at[b, i], acc.at[dstblk.at[b, i]],
                             ssem, add=True)
            return cc

        lax.fori_loop(0, GRP, wf, 0)

    def drain_scatters(b, ssem):
        def dr(i, cc):
            pltpu.make_async_copy(rowbuf.at[b, i], acc.at[dstblk.at[b, i]],
                                  ssem).wait()
            return cc

        lax.fori_loop(0, GRP, dr, 0)

    def step_mid(g, b, gsem_b, ssem_b, gsem_o, ssem_o):
        @pl.when(g + 1 < NG)
        def _():
            fire_idx(g + 1, 1 - b)

        wait_fire_scatters(b, gsem_b, ssem_b)
        drain_scatters(1 - b, ssem_o)

        @pl.when(g + 1 < NG)
        def _():
            wait_idx(g + 1, 1 - b)
            fire_gathers(1 - b, gsem_o)

    pltpu.sync_copy(src_r.at[pl.ds(base, GRP)], idxblk.at[0])
    pltpu.sync_copy(dst_r.at[pl.ds(base, GRP)], dstblk.at[0])
    fire_gathers(0, ga)
    fire_idx(1, 1)
    wait_fire_scatters(0, ga, sa)
    wait_idx(1, 1)
    fire_gathers(1, gb)

    def pair(gg, cc):
        g1 = 1 + 2 * gg
        step_mid(g1, 1, gb, sb, ga, sa)
        step_mid(g1 + 1, 0, ga, sa, gb, sb)
        return cc

    lax.fori_loop(0, (NG - 1) // 2, pair, 0)
    drain_scatters(0, sa)


def _dump(t, acc, out_ref, lead=None):
    sl = pl.ds(t * RPT, RPT)
    dst = out_ref.at[lead, sl] if lead is not None else out_ref.at[sl]
    pltpu.sync_copy(acc.at[sl], dst)


def _seg_phases(c, t, acc, zbuf, idxblk, dstblk, rowbuf, sems,
                src_r, dst_r, tables, out_ref):
    for p in range(2):
        _zero_acc(t, acc, zbuf)
        plsc.subcore_barrier()

        @pl.when(c == 0)
        def _():
            _scatter_pipeline(t, acc, idxblk, dstblk, rowbuf, src_r, dst_r,
                              tables[p], *sems)

        @pl.when(c == 1)
        def _():
            _scatter_pipeline(t, acc, idxblk, dstblk, rowbuf, src_r, dst_r,
                              tables[2 + p], *sems)

        plsc.subcore_barrier()

        @pl.when(c == 0)
        def _():
            _dump(t, acc, out_ref, lead=p)

        @pl.when(c == 1)
        def _():
            _dump(t, acc, out_ref, lead=2 + p)

        plsc.subcore_barrier()


_SC_SCRATCH = [
    pltpu.VMEM_SHARED((NPAD, Q), jnp.float32),
    pltpu.VMEM((ZB, Q), jnp.float32),
    pltpu.VMEM((CH, Q), jnp.float32),
    pltpu.VMEM((2, GRP, CH), jnp.int32),
    pltpu.VMEM((2, GRP, CH), jnp.int32),
    pltpu.VMEM((2, GRP, CH, Q), jnp.float32),
]

_SEM5 = (pltpu.SemaphoreType.DMA,) * 5


def _layer0_body(src_sc, dst_sc, src_cs, dst_cs,
                 ts0, ts1, ts2, ts3, tc0, tc1, tc2, tc3,
                 cnt_code, cnt_stay, s_code, s_stay,
                 acc, zbuf, obuf, idxblk, dstblk, rowbuf):
    def scoped(*sems):
        _layer0_inner(src_sc, dst_sc, src_cs, dst_cs,
                      ts0, ts1, ts2, ts3, tc0, tc1, tc2, tc3,
                      cnt_code, cnt_stay, s_code, s_stay,
                      acc, zbuf, obuf, idxblk, dstblk, rowbuf, sems)

    pl.run_scoped(scoped, *_SEM5)


def _layer0_inner(src_sc, dst_sc, src_cs, dst_cs,
                  ts0, ts1, ts2, ts3, tc0, tc1, tc2, tc3,
                  cnt_code, cnt_stay, s_code, s_stay,
                  acc, zbuf, obuf, idxblk, dstblk, rowbuf, sems):
    c = lax.axis_index("c")
    t = lax.axis_index("s")
    _fill(zbuf, ZB, 0.0)
    _fill(obuf, CH, 1.0)

    _zero_acc(t, acc, zbuf)
    plsc.subcore_barrier()

    @pl.when(c == 0)
    def _():
        _count_loop(t, acc, dstblk, obuf, dst_sc, sems[2])

    @pl.when(c == 1)
    def _():
        _count_loop(t, acc, dstblk, obuf, dst_cs, sems[2])

    plsc.subcore_barrier()

    @pl.when(c == 0)
    def _():
        _dump(t, acc, cnt_code)

    @pl.when(c == 1)
    def _():
        _dump(t, acc, cnt_stay)

    plsc.subcore_barrier()

    _seg_phases(c, t, acc, zbuf, idxblk, dstblk, rowbuf, sems,
                src_sc, dst_sc, (ts0, ts1, ts2, ts3), s_code)

    _seg_phases(c, t, acc, zbuf, idxblk, dstblk, rowbuf, sems,
                src_cs, dst_cs, (tc0, tc1, tc2, tc3), s_stay)


def _layer1_body(src_cs, dst_cs, tc0, tc1, tc2, tc3, s_stay,
                 acc, zbuf, idxblk, dstblk, rowbuf):
    def scoped(*sems):
        c = lax.axis_index("c")
        t = lax.axis_index("s")
        _fill(zbuf, ZB, 0.0)
        _seg_phases(c, t, acc, zbuf, idxblk, dstblk, rowbuf, sems,
                    src_cs, dst_cs, (tc0, tc1, tc2, tc3), s_stay)

    pl.run_scoped(scoped, *_SEM5)


def _sc_layer0(src_sc, dst_sc, src_cs, dst_cs, ts, tc):
    f = pl.kernel(
        _layer0_body,
        out_type=[
            jax.ShapeDtypeStruct((NPAD, Q), jnp.float32),
            jax.ShapeDtypeStruct((NPAD, Q), jnp.float32),
            jax.ShapeDtypeStruct((4, NPAD, Q), jnp.float32),
            jax.ShapeDtypeStruct((4, NPAD, Q), jnp.float32),
        ],
        mesh=plsc.VectorSubcoreMesh(core_axis_name="c", subcore_axis_name="s"),
        scratch_types=_SC_SCRATCH,
        compiler_params=pltpu.CompilerParams(use_tc_tiling_on_sc=False),
    )
    return f(src_sc, dst_sc, src_cs, dst_cs, *ts, *tc)


def _sc_layer1(src_cs, dst_cs, tc):
    f = pl.kernel(
        _layer1_body,
        out_type=jax.ShapeDtypeStruct((4, NPAD, Q), jnp.float32),
        mesh=plsc.VectorSubcoreMesh(core_axis_name="c", subcore_axis_name="s"),
        scratch_types=[_SC_SCRATCH[0], _SC_SCRATCH[1]] + _SC_SCRATCH[3:],
        compiler_params=pltpu.CompilerParams(use_tc_tiling_on_sc=False),
    )
    return f(src_cs, dst_cs, *tc)


def _proj_body(x_ref, w_ref, b_ref, o0, o1, o2, o3):
    y = jnp.dot(x_ref[...], w_ref[...], preferred_element_type=jnp.float32)
    y = jnp.maximum(y + b_ref[...], 0.0)
    for q, o in enumerate((o0, o1, o2, o3)):
        o[...] = y[:, q * Q:(q + 1) * Q]


def _proj(x, w, b):
    d_in = x.shape[1]
    return pl.pallas_call(
        _proj_body,
        grid=(N // BLK,),
        in_specs=[
            pl.BlockSpec((BLK, d_in), lambda i: (i, 0)),
            pl.BlockSpec((d_in, H), lambda i: (0, 0)),
            pl.BlockSpec((1, H), lambda i: (0, 0)),
        ],
        out_specs=[pl.BlockSpec((BLK, Q), lambda i: (i, 0))] * 4,
        out_shape=[jax.ShapeDtypeStruct((N, Q), jnp.float32)] * 4,
    )(x, w, b.reshape(1, H))


def _sage_update(s_ref, cnt_ref, h_refs, wl_ref, bl_ref, wr_ref, br_ref,
                 g_ref, be_ref):
    cnt = jnp.maximum(cnt_ref[:, 0:1], 1.0)
    wl = wl_ref[...]
    wr = wr_ref[...]
    out = bl_ref[...] + br_ref[...]
    for q in range(4):
        out += jnp.dot(s_ref[q] / cnt, wl[q * Q:(q + 1) * Q],
                       preferred_element_type=jnp.float32)
        out += jnp.dot(h_refs[q][...], wr[q * Q:(q + 1) * Q],
                       preferred_element_type=jnp.float32)
    nrm = jnp.sqrt(jnp.sum(out * out, axis=-1, keepdims=True))
    out = out / jnp.maximum(nrm, 1e-12)
    r = jnp.maximum(out, 0.0)
    m = jnp.mean(r, axis=-1, keepdims=True)
    v = jnp.mean((r - m) ** 2, axis=-1, keepdims=True)
    return (r - m) / jnp.sqrt(v + 1e-5) * g_ref[...] + be_ref[...]


def _update_body(s_ref, cnt_ref, h0, h1, h2, h3, wl_ref, bl_ref, wr_ref,
                 br_ref, g_ref, be_ref, o0, o1, o2, o3):
    h = _sage_update(s_ref, cnt_ref, (h0, h1, h2, h3), wl_ref, bl_ref,
                     wr_ref, br_ref, g_ref, be_ref)
    for q, o in enumerate((o0, o1, o2, o3)):
        o[...] = h[:, q * Q:(q + 1) * Q]


def _final_body(s_ref, cnt_ref, h0, h1, h2, h3, wl_ref, bl_ref, wr_ref,
                br_ref, g_ref, be_ref, wc_ref, bc_ref, logits_ref):
    h = _sage_update(s_ref, cnt_ref, (h0, h1, h2, h3), wl_ref, bl_ref,
                     wr_ref, br_ref, g_ref, be_ref)
    logits_ref[...] = (jnp.dot(h, wc_ref[...], preferred_element_type=jnp.float32)
                       + bc_ref[...])


_W_SPECS = [
    pl.BlockSpec((H, H), lambda i: (0, 0)),
    pl.BlockSpec((1, H), lambda i: (0, 0)),
    pl.BlockSpec((H, H), lambda i: (0, 0)),
    pl.BlockSpec((1, H), lambda i: (0, 0)),
    pl.BlockSpec((1, H), lambda i: (0, 0)),
    pl.BlockSpec((1, H), lambda i: (0, 0)),
]

_IN_SPECS = [
    pl.BlockSpec((4, BLK, Q), lambda i: (0, i, 0)),
    pl.BlockSpec((BLK, Q), lambda i: (i, 0)),
    pl.BlockSpec((BLK, Q), lambda i: (i, 0)),
    pl.BlockSpec((BLK, Q), lambda i: (i, 0)),
    pl.BlockSpec((BLK, Q), lambda i: (i, 0)),
    pl.BlockSpec((BLK, Q), lambda i: (i, 0)),
]


def _update(s, cnt, h, wl, bl, wr, br, g, be):
    return pl.pallas_call(
        _update_body,
        grid=(N // BLK,),
        in_specs=_IN_SPECS + _W_SPECS,
        out_specs=[pl.BlockSpec((BLK, Q), lambda i: (i, 0))] * 4,
        out_shape=[jax.ShapeDtypeStruct((N, Q), jnp.float32)] * 4,
    )(s, cnt, *h, wl, bl.reshape(1, -1), wr, br.reshape(1, -1),
      g.reshape(1, -1), be.reshape(1, -1))


def _final(s, cnt, h, wl, bl, wr, br, g, be, wc, bc):
    n_cls = wc.shape[1]
    return pl.pallas_call(
        _final_body,
        grid=(N // BLK,),
        in_specs=_IN_SPECS + _W_SPECS + [
            pl.BlockSpec((H, n_cls), lambda i: (0, 0)),
            pl.BlockSpec((1, n_cls), lambda i: (0, 0)),
        ],
        out_specs=pl.BlockSpec((BLK, n_cls), lambda i: (i, 0)),
        out_shape=jax.ShapeDtypeStruct((N, n_cls), jnp.float32),
    )(s, cnt, *h, wl, bl.reshape(1, -1), wr, br.reshape(1, -1),
      g.reshape(1, -1), be.reshape(1, -1), wc, bc.reshape(1, -1))


def kernel(x_stay, x_code, ei_sc, ei_cs, W_in_stay, b_in_stay, W_in_code,
           b_in_code, Wl0_sc, bl0_sc, Wr0_sc, br0_sc, Wl0_cs, bl0_cs, Wr0_cs,
           br0_cs, g0, be0, Wl1_sc, bl1_sc, Wr1_sc, br1_sc, Wl1_cs, bl1_cs,
           Wr1_cs, br1_cs, g1, be1, Wc, bc):
    src_sc = ei_sc[0].reshape(E // CH, CH)
    dst_sc = ei_sc[1].reshape(E // CH, CH)
    src_cs = ei_cs[0].reshape(E // CH, CH)
    dst_cs = ei_cs[1].reshape(E // CH, CH)

    hs = _proj(x_stay, W_in_stay, b_in_stay)
    hc = _proj(x_code, W_in_code, b_in_code)

    cnt_code, cnt_stay, s_code, s_stay = _sc_layer0(
        src_sc, dst_sc, src_cs, dst_cs, hs, hc)

    hc1 = _update(s_code, cnt_code, hc, Wl0_sc, bl0_sc, Wr0_sc, br0_sc,
                  g0, be0)
    hs1 = _update(s_stay, cnt_stay, hs, Wl0_cs, bl0_cs, Wr0_cs, br0_cs,
                  g0, be0)

    s_stay1 = _sc_layer1(src_cs, dst_cs, hc1)

    return _final(s_stay1, cnt_stay, hs1, Wl1_cs, bl1_cs, Wr1_cs, br1_cs,
                  g1, be1, Wc, bc)


# speedup vs baseline: 10.3758x; 1.4412x over previous
import jax
import jax.numpy as jnp
from jax import lax
from jax.experimental import pallas as pl
from jax.experimental.pallas import tpu as pltpu
from jax.experimental.pallas import tpu_sc as plsc

N = 50000
E = 800000
H = 64
Q = 16
CH = 80
NV = CH // 16
NTILES = 16
CPT = E // CH // NTILES
GRP = 25
NG = CPT // GRP
NPAD = 51200
RPT = NPAD // NTILES
ZB = 200
BLK = 2048
GRID = NPAD // BLK


def _fill(buf, nrows, val):
    v = jnp.full((16,), val, jnp.float32)

    def body(r, carry):
        buf[r, pl.ds(0, 16)] = v
        return carry

    lax.fori_loop(0, nrows, body, 0)


def _zero_acc(t, acc, zbuf):
    for k in range(RPT // ZB):
        pltpu.sync_copy(zbuf, acc.at[pl.ds(t * RPT + k * ZB, ZB)])


def _count_loop(t, acc, dstblk, obuf, dst_r, ssem):
    base = t * CPT

    def group(g, carry):
        pltpu.sync_copy(dst_r.at[pl.ds(base + g * GRP, GRP)], dstblk.at[0])

        def fire(i, cc):
            pltpu.async_copy(obuf, acc.at[dstblk.at[0, i]], ssem, add=True)
            return cc

        lax.fori_loop(0, GRP, fire, 0)

        def drain(i, cc):
            pltpu.make_async_copy(obuf, acc.at[dstblk.at[0, i]],
                                  ssem).wait()
            return cc

        lax.fori_loop(0, GRP, drain, 0)
        return carry

    lax.fori_loop(0, NG, group, 0)


def _scatter_pipeline(t, acc, idxblk, dstblk, rowbuf, src_r, dst_r, table,
                      qoff, ga, gb, sa, sb, isem):
    base = t * CPT

    def fire_idx(g, b):
        sl = pl.ds(base + g * GRP, GRP)
        pltpu.async_copy(src_r.at[sl], idxblk.at[b], isem)
        pltpu.async_copy(dst_r.at[sl], dstblk.at[b], isem)

    def wait_idx(g, b):
        sl = pl.ds(base + g * GRP, GRP)
        pltpu.make_async_copy(src_r.at[sl], idxblk.at[b], isem).wait()
        pltpu.make_async_copy(dst_r.at[sl], dstblk.at[b], isem).wait()

    def xform_idx(b):
        def body(m, cc):
            i = m // NV
            l = m % NV
            v = idxblk[b, i, pl.ds(l * 16, 16)]
            idxblk[b, i, pl.ds(l * 16, 16)] = v * 8 + qoff
            return cc

        lax.fori_loop(0, GRP * NV, body, 0)

    def fire_gathers(b, gsem):
        def fire(i, cc):
            pltpu.async_copy(table.at[idxblk.at[b, i]], rowbuf.at[b, i],
                             gsem)
            return cc

        lax.fori_loop(0, GRP, fire, 0)

    def wait_fire_scatters(b, gsem, ssem):
        def wf(i, cc):
            pltpu.make_async_copy(table.at[idxblk.at[b, i]],
                                  rowbuf.at[b, i], gsem).wait()
            pltpu.async_copy(rowbuf.at[b, i], acc.at[dstblk.at[b, i]],
                             ssem, add=True)
            return cc

        lax.fori_loop(0, GRP, wf, 0)

    def drain_scatters(b, ssem):
        def dr(i, cc):
            pltpu.make_async_copy(rowbuf.at[b, i], acc.at[dstblk.at[b, i]],
                                  ssem).wait()
            return cc

        lax.fori_loop(0, GRP, dr, 0)

    def step_mid(g, b, gsem_b, ssem_b, gsem_o, ssem_o):
        @pl.when(g + 1 < NG)
        def _():
            fire_idx(g + 1, 1 - b)

        wait_fire_scatters(b, gsem_b, ssem_b)
        drain_scatters(1 - b, ssem_o)

        @pl.when(g + 1 < NG)
        def _():
            wait_idx(g + 1, 1 - b)
            xform_idx(1 - b)
            fire_gathers(1 - b, gsem_o)

    pltpu.sync_copy(src_r.at[pl.ds(base, GRP)], idxblk.at[0])
    pltpu.sync_copy(dst_r.at[pl.ds(base, GRP)], dstblk.at[0])
    xform_idx(0)
    fire_gathers(0, ga)
    fire_idx(1, 1)
    wait_fire_scatters(0, ga, sa)
    wait_idx(1, 1)
    xform_idx(1)
    fire_gathers(1, gb)

    def pair(gg, cc):
        g1 = 1 + 2 * gg
        step_mid(g1, 1, gb, sb, ga, sa)
        step_mid(g1 + 1, 0, ga, sa, gb, sb)
        return cc

    lax.fori_loop(0, (NG - 1) // 2, pair, 0)
    drain_scatters(0, sa)


def _dump(t, acc, out_ref, q):
    sl = pl.ds(t * RPT, RPT)
    pltpu.sync_copy(acc.at[sl], out_ref.at[sl, pl.ds(Q * q, Q)])


def _seg_phases(c, t, acc, zbuf, idxblk, dstblk, rowbuf, sems,
                src_r, dst_r, table, out_ref):
    for p in range(2):
        _zero_acc(t, acc, zbuf)
        plsc.subcore_barrier()

        @pl.when(c == 0)
        def _():
            _scatter_pipeline(t, acc, idxblk, dstblk, rowbuf, src_r, dst_r,
                              table, p, *sems)

        @pl.when(c == 1)
        def _():
            _scatter_pipeline(t, acc, idxblk, dstblk, rowbuf, src_r, dst_r,
                              table, 2 + p, *sems)

        plsc.subcore_barrier()

        @pl.when(c == 0)
        def _():
            _dump(t, acc, out_ref, p)

        @pl.when(c == 1)
        def _():
            _dump(t, acc, out_ref, 2 + p)

        plsc.subcore_barrier()


_SC_SCRATCH = [
    pltpu.VMEM_SHARED((NPAD, Q), jnp.float32),
    pltpu.VMEM((ZB, Q), jnp.float32),
    pltpu.VMEM((CH, Q), jnp.float32),
    pltpu.VMEM((2, GRP, CH), jnp.int32),
    pltpu.VMEM((2, GRP, CH), jnp.int32),
    pltpu.VMEM((2, GRP, CH, Q), jnp.float32),
]

_SEM5 = (pltpu.SemaphoreType.DMA,) * 5


def _layer0_body(src_sc, dst_sc, src_cs, dst_cs, tstay, tcode,
                 s_code, s_stay,
                 acc, zbuf, obuf, idxblk, dstblk, rowbuf):
    def scoped(*sems):
        c = lax.axis_index("c")
        t = lax.axis_index("s")
        _fill(zbuf, ZB, 0.0)
        _fill(obuf, CH, 1.0)

        _zero_acc(t, acc, zbuf)
        plsc.subcore_barrier()

        @pl.when(c == 0)
        def _():
            _count_loop(t, acc, dstblk, obuf, dst_sc, sems[2])

        @pl.when(c == 1)
        def _():
            _count_loop(t, acc, dstblk, obuf, dst_cs, sems[2])

        plsc.subcore_barrier()

        @pl.when(c == 0)
        def _():
            _dump(t, acc, s_code, 4)

        @pl.when(c == 1)
        def _():
            _dump(t, acc, s_stay, 4)

        plsc.subcore_barrier()

        _seg_phases(c, t, acc, zbuf, idxblk, dstblk, rowbuf, sems,
                    src_sc, dst_sc, tstay, s_code)

        _seg_phases(c, t, acc, zbuf, idxblk, dstblk, rowbuf, sems,
                    src_cs, dst_cs, tcode, s_stay)

    pl.run_scoped(scoped, *_SEM5)


def _layer1_body(src_cs, dst_cs, tcode, s_stay,
                 acc, zbuf, idxblk, dstblk, rowbuf):
    def scoped(*sems):
        c = lax.axis_index("c")
        t = lax.axis_index("s")
        _fill(zbuf, ZB, 0.0)
        _seg_phases(c, t, acc, zbuf, idxblk, dstblk, rowbuf, sems,
                    src_cs, dst_cs, tcode, s_stay)

    pl.run_scoped(scoped, *_SEM5)


def _sc_layer0(src_sc, dst_sc, src_cs, dst_cs, tstay, tcode):
    f = pl.kernel(
        _layer0_body,
        out_type=[
            jax.ShapeDtypeStruct((NPAD, 128), jnp.float32),
            jax.ShapeDtypeStruct((NPAD, 128), jnp.float32),
        ],
        mesh=plsc.VectorSubcoreMesh(core_axis_name="c", subcore_axis_name="s"),
        scratch_types=_SC_SCRATCH,
        compiler_params=pltpu.CompilerParams(use_tc_tiling_on_sc=False),
    )
    return f(src_sc, dst_sc, src_cs, dst_cs, tstay, tcode)


def _sc_layer1(src_cs, dst_cs, tcode):
    f = pl.kernel(
        _layer1_body,
        out_type=jax.ShapeDtypeStruct((NPAD, 128), jnp.float32),
        mesh=plsc.VectorSubcoreMesh(core_axis_name="c", subcore_axis_name="s"),
        scratch_types=[_SC_SCRATCH[0], _SC_SCRATCH[1]] + _SC_SCRATCH[3:],
        compiler_params=pltpu.CompilerParams(use_tc_tiling_on_sc=False),
    )
    return f(src_cs, dst_cs, tcode)


def _proj_body(x_ref, w_ref, b_ref, o_ref):
    y = jnp.dot(x_ref[...], w_ref[...], preferred_element_type=jnp.float32)
    y = jnp.maximum(y + b_ref[...], 0.0)
    o_ref[:, :H] = y
    o_ref[:, H:] = jnp.zeros((BLK, 128 - H), jnp.float32)


def _proj(x, w, b):
    d_in = x.shape[1]
    return pl.pallas_call(
        _proj_body,
        grid=(GRID,),
        in_specs=[
            pl.BlockSpec((BLK, d_in), lambda i: (i, 0)),
            pl.BlockSpec((d_in, H), lambda i: (0, 0)),
            pl.BlockSpec((1, H), lambda i: (0, 0)),
        ],
        out_specs=pl.BlockSpec((BLK, 128), lambda i: (i, 0)),
        out_shape=jax.ShapeDtypeStruct((NPAD, 128), jnp.float32),
    )(x, w, b.reshape(1, H))


def _sage_core(s, cnt, h, wl_ref, bl_ref, wr_ref, br_ref, g_ref, be_ref):
    agg = s / cnt
    out = (jnp.dot(agg, wl_ref[...], preferred_element_type=jnp.float32)
           + bl_ref[...]
           + jnp.dot(h, wr_ref[...], preferred_element_type=jnp.float32)
           + br_ref[...])
    nrm = jnp.sqrt(jnp.sum(out * out, axis=-1, keepdims=True))
    out = out / jnp.maximum(nrm, 1e-12)
    r = jnp.maximum(out, 0.0)
    m = jnp.mean(r, axis=-1, keepdims=True)
    v = jnp.mean((r - m) ** 2, axis=-1, keepdims=True)
    return (r - m) / jnp.sqrt(v + 1e-5) * g_ref[...] + be_ref[...]


def _update_body(s_ref, h_ref, wl_ref, bl_ref, wr_ref, br_ref, g_ref,
                 be_ref, o_ref):
    cnt = jnp.maximum(s_ref[:, H:H + 1], 1.0)
    hn = _sage_core(s_ref[:, :H], cnt, h_ref[:, :H], wl_ref, bl_ref,
                    wr_ref, br_ref, g_ref, be_ref)
    o_ref[:, :H] = hn
    o_ref[:, H:] = jnp.zeros((BLK, 128 - H), jnp.float32)


def _final_body(s1_ref, s0_ref, h_ref, wl_ref, bl_ref, wr_ref, br_ref,
                g_ref, be_ref, wc_ref, bc_ref, logits_ref):
    cnt = jnp.maximum(s0_ref[:, H:H + 1], 1.0)
    hn = _sage_core(s1_ref[:, :H], cnt, h_ref[:, :H], wl_ref, bl_ref,
                    wr_ref, br_ref, g_ref, be_ref)
    logits_ref[...] = (jnp.dot(hn, wc_ref[...],
                               preferred_element_type=jnp.float32)
                       + bc_ref[...])


_W_SPECS = [
    pl.BlockSpec((H, H), lambda i: (0, 0)),
    pl.BlockSpec((1, H), lambda i: (0, 0)),
    pl.BlockSpec((H, H), lambda i: (0, 0)),
    pl.BlockSpec((1, H), lambda i: (0, 0)),
    pl.BlockSpec((1, H), lambda i: (0, 0)),
    pl.BlockSpec((1, H), lambda i: (0, 0)),
]

_B128 = pl.BlockSpec((BLK, 128), lambda i: (i, 0))


def _update(s, h, wl, bl, wr, br, g, be):
    return pl.pallas_call(
        _update_body,
        grid=(GRID,),
        in_specs=[_B128, _B128] + _W_SPECS,
        out_specs=_B128,
        out_shape=jax.ShapeDtypeStruct((NPAD, 128), jnp.float32),
    )(s, h, wl, bl.reshape(1, -1), wr, br.reshape(1, -1),
      g.reshape(1, -1), be.reshape(1, -1))


def _final(s1, s0, h, wl, bl, wr, br, g, be, wc, bc):
    n_cls = wc.shape[1]
    return pl.pallas_call(
        _final_body,
        grid=(GRID,),
        in_specs=[_B128, _B128, _B128] + _W_SPECS + [
            pl.BlockSpec((H, n_cls), lambda i: (0, 0)),
            pl.BlockSpec((1, n_cls), lambda i: (0, 0)),
        ],
        out_specs=pl.BlockSpec((BLK, n_cls), lambda i: (i, 0)),
        out_shape=jax.ShapeDtypeStruct((N, n_cls), jnp.float32),
    )(s1, s0, h, wl, bl.reshape(1, -1), wr, br.reshape(1, -1),
      g.reshape(1, -1), be.reshape(1, -1), wc, bc.reshape(1, -1))


def kernel(x_stay, x_code, ei_sc, ei_cs, W_in_stay, b_in_stay, W_in_code,
           b_in_code, Wl0_sc, bl0_sc, Wr0_sc, br0_sc, Wl0_cs, bl0_cs, Wr0_cs,
           br0_cs, g0, be0, Wl1_sc, bl1_sc, Wr1_sc, br1_sc, Wl1_cs, bl1_cs,
           Wr1_cs, br1_cs, g1, be1, Wc, bc):
    src_sc = ei_sc[0].reshape(E // CH, CH)
    dst_sc = ei_sc[1].reshape(E // CH, CH)
    src_cs = ei_cs[0].reshape(E // CH, CH)
    dst_cs = ei_cs[1].reshape(E // CH, CH)

    hs = _proj(x_stay, W_in_stay, b_in_stay)
    hc = _proj(x_code, W_in_code, b_in_code)

    s_code, s_stay = _sc_layer0(src_sc, dst_sc, src_cs, dst_cs,
                                hs.reshape(8 * NPAD, Q),
                                hc.reshape(8 * NPAD, Q))

    hc1 = _update(s_code, hc, Wl0_sc, bl0_sc, Wr0_sc, br0_sc, g0, be0)
    hs1 = _update(s_stay, hs, Wl0_cs, bl0_cs, Wr0_cs, br0_cs, g0, be0)

    s_stay1 = _sc_layer1(src_cs, dst_cs, hc1.reshape(8 * NPAD, Q))

    return _final(s_stay1, s_stay, hs1, Wl1_cs, bl1_cs, Wr1_cs, br1_cs,
                  g1, be1, Wc, bc)


# speedup vs baseline: 10.3780x; 1.0002x over previous
import jax
import jax.numpy as jnp
from jax import lax
from jax.experimental import pallas as pl
from jax.experimental.pallas import tpu as pltpu
from jax.experimental.pallas import tpu_sc as plsc

N = 50000
E = 800000
H = 64
Q = 16
CH = 80
NV = CH // 16
NTILES = 16
CPT = E // CH // NTILES
GRP = 25
NG = CPT // GRP
NPAD = 51200
RPT = NPAD // NTILES
ZB = 200
BLK = 2048
GRID = NPAD // BLK


def _fill(buf, nrows, val):
    v = jnp.full((16,), val, jnp.float32)

    def body(r, carry):
        buf[r, pl.ds(0, 16)] = v
        return carry

    lax.fori_loop(0, nrows, body, 0)


def _zero_acc(t, acc, zbuf):
    for k in range(RPT // ZB):
        pltpu.sync_copy(zbuf, acc.at[pl.ds(t * RPT + k * ZB, ZB)])


def _count_loop(t, acc, dstblk, obuf, dst_r, ssem):
    base = t * CPT

    def group(g, carry):
        pltpu.sync_copy(dst_r.at[pl.ds(base + g * GRP, GRP)], dstblk.at[0])

        def fire(i, cc):
            pltpu.async_copy(obuf, acc.at[dstblk.at[0, i]], ssem, add=True)
            return cc

        lax.fori_loop(0, GRP, fire, 0)

        def drain(i, cc):
            pltpu.make_async_copy(obuf, acc.at[dstblk.at[0, i]],
                                  ssem).wait()
            return cc

        lax.fori_loop(0, GRP, drain, 0)
        return carry

    lax.fori_loop(0, NG, group, 0)


def _scatter_pipeline(t, acc, idxblk, dstblk, rowbuf, src_r, dst_r, table,
                      qoff, ga, gb, sa, sb, isem):
    base = t * CPT

    def fire_idx(g, b):
        sl = pl.ds(base + g * GRP, GRP)
        pltpu.async_copy(src_r.at[sl], idxblk.at[b], isem)
        pltpu.async_copy(dst_r.at[sl], dstblk.at[b], isem)

    def wait_idx(g, b):
        sl = pl.ds(base + g * GRP, GRP)
        pltpu.make_async_copy(src_r.at[sl], idxblk.at[b], isem).wait()
        pltpu.make_async_copy(dst_r.at[sl], dstblk.at[b], isem).wait()

    def xform_idx(b):
        def body(m, cc):
            i = m // NV
            l = m % NV
            v = idxblk[b, i, pl.ds(l * 16, 16)]
            idxblk[b, i, pl.ds(l * 16, 16)] = v * 8 + qoff
            return cc

        lax.fori_loop(0, GRP * NV, body, 0)

    def fire_gathers(b, gsem):
        def fire(i, cc):
            pltpu.async_copy(table.at[idxblk.at[b, i]],
                             rowbuf.at[b, pl.ds(i * CH, CH)], gsem)
            return cc

        lax.fori_loop(0, GRP, fire, 0)

    def wait_fire_scatters(b, gsem, ssem):
        pltpu.make_async_copy(table.at[pl.ds(0, GRP * CH)], rowbuf.at[b],
                              gsem).wait()

        def fire(i, cc):
            pltpu.async_copy(rowbuf.at[b, pl.ds(i * CH, CH)],
                             acc.at[dstblk.at[b, i]], ssem, add=True)
            return cc

        lax.fori_loop(0, GRP, fire, 0)

    def drain_scatters(b, ssem):
        pltpu.make_async_copy(rowbuf.at[b], acc.at[pl.ds(0, GRP * CH)],
                              ssem).wait()

    def step_mid(g, b, gsem_b, ssem_b, gsem_o, ssem_o):
        @pl.when(g + 1 < NG)
        def _():
            fire_idx(g + 1, 1 - b)

        wait_fire_scatters(b, gsem_b, ssem_b)
        drain_scatters(1 - b, ssem_o)

        @pl.when(g + 1 < NG)
        def _():
            wait_idx(g + 1, 1 - b)
            xform_idx(1 - b)
            fire_gathers(1 - b, gsem_o)

    pltpu.sync_copy(src_r.at[pl.ds(base, GRP)], idxblk.at[0])
    pltpu.sync_copy(dst_r.at[pl.ds(base, GRP)], dstblk.at[0])
    xform_idx(0)
    fire_gathers(0, ga)
    fire_idx(1, 1)
    wait_fire_scatters(0, ga, sa)
    wait_idx(1, 1)
    xform_idx(1)
    fire_gathers(1, gb)

    def pair(gg, cc):
        g1 = 1 + 2 * gg
        step_mid(g1, 1, gb, sb, ga, sa)
        step_mid(g1 + 1, 0, ga, sa, gb, sb)
        return cc

    lax.fori_loop(0, (NG - 1) // 2, pair, 0)
    drain_scatters(0, sa)


def _dump(t, acc, out_ref, q):
    sl = pl.ds(t * RPT, RPT)
    pltpu.sync_copy(acc.at[sl], out_ref.at[sl, pl.ds(Q * q, Q)])


def _seg_phases(c, t, acc, zbuf, idxblk, dstblk, rowbuf, sems,
                src_r, dst_r, table, out_ref):
    for p in range(2):
        _zero_acc(t, acc, zbuf)
        plsc.subcore_barrier()

        @pl.when(c == 0)
        def _():
            _scatter_pipeline(t, acc, idxblk, dstblk, rowbuf, src_r, dst_r,
                              table, p, *sems)

        @pl.when(c == 1)
        def _():
            _scatter_pipeline(t, acc, idxblk, dstblk, rowbuf, src_r, dst_r,
                              table, 2 + p, *sems)

        plsc.subcore_barrier()

        @pl.when(c == 0)
        def _():
            _dump(t, acc, out_ref, p)

        @pl.when(c == 1)
        def _():
            _dump(t, acc, out_ref, 2 + p)

        plsc.subcore_barrier()


_SC_SCRATCH = [
    pltpu.VMEM_SHARED((NPAD, Q), jnp.float32),
    pltpu.VMEM((ZB, Q), jnp.float32),
    pltpu.VMEM((CH, Q), jnp.float32),
    pltpu.VMEM((2, GRP, CH), jnp.int32),
    pltpu.VMEM((2, GRP, CH), jnp.int32),
    pltpu.VMEM((2, GRP * CH, Q), jnp.float32),
]

_SEM5 = (pltpu.SemaphoreType.DMA,) * 5


def _layer0_body(src_sc, dst_sc, src_cs, dst_cs, tstay, tcode,
                 s_code, s_stay,
                 acc, zbuf, obuf, idxblk, dstblk, rowbuf):
    def scoped(*sems):
        c = lax.axis_index("c")
        t = lax.axis_index("s")
        _fill(zbuf, ZB, 0.0)
        _fill(obuf, CH, 1.0)

        _zero_acc(t, acc, zbuf)
        plsc.subcore_barrier()

        @pl.when(c == 0)
        def _():
            _count_loop(t, acc, dstblk, obuf, dst_sc, sems[2])

        @pl.when(c == 1)
        def _():
            _count_loop(t, acc, dstblk, obuf, dst_cs, sems[2])

        plsc.subcore_barrier()

        @pl.when(c == 0)
        def _():
            _dump(t, acc, s_code, 4)

        @pl.when(c == 1)
        def _():
            _dump(t, acc, s_stay, 4)

        plsc.subcore_barrier()

        _seg_phases(c, t, acc, zbuf, idxblk, dstblk, rowbuf, sems,
                    src_sc, dst_sc, tstay, s_code)

        _seg_phases(c, t, acc, zbuf, idxblk, dstblk, rowbuf, sems,
                    src_cs, dst_cs, tcode, s_stay)

    pl.run_scoped(scoped, *_SEM5)


def _layer1_body(src_cs, dst_cs, tcode, s_stay,
                 acc, zbuf, idxblk, dstblk, rowbuf):
    def scoped(*sems):
        c = lax.axis_index("c")
        t = lax.axis_index("s")
        _fill(zbuf, ZB, 0.0)
        _seg_phases(c, t, acc, zbuf, idxblk, dstblk, rowbuf, sems,
                    src_cs, dst_cs, tcode, s_stay)

    pl.run_scoped(scoped, *_SEM5)


def _sc_layer0(src_sc, dst_sc, src_cs, dst_cs, tstay, tcode):
    f = pl.kernel(
        _layer0_body,
        out_type=[
            jax.ShapeDtypeStruct((NPAD, 128), jnp.float32),
            jax.ShapeDtypeStruct((NPAD, 128), jnp.float32),
        ],
        mesh=plsc.VectorSubcoreMesh(core_axis_name="c", subcore_axis_name="s"),
        scratch_types=_SC_SCRATCH,
        compiler_params=pltpu.CompilerParams(use_tc_tiling_on_sc=False),
    )
    return f(src_sc, dst_sc, src_cs, dst_cs, tstay, tcode)


def _sc_layer1(src_cs, dst_cs, tcode):
    f = pl.kernel(
        _layer1_body,
        out_type=jax.ShapeDtypeStruct((NPAD, 128), jnp.float32),
        mesh=plsc.VectorSubcoreMesh(core_axis_name="c", subcore_axis_name="s"),
        scratch_types=[_SC_SCRATCH[0], _SC_SCRATCH[1]] + _SC_SCRATCH[3:],
        compiler_params=pltpu.CompilerParams(use_tc_tiling_on_sc=False),
    )
    return f(src_cs, dst_cs, tcode)


def _proj_body(x_ref, w_ref, b_ref, o_ref):
    y = jnp.dot(x_ref[...], w_ref[...], preferred_element_type=jnp.float32)
    y = jnp.maximum(y + b_ref[...], 0.0)
    o_ref[:, :H] = y
    o_ref[:, H:] = jnp.zeros((BLK, 128 - H), jnp.float32)


def _proj(x, w, b):
    d_in = x.shape[1]
    return pl.pallas_call(
        _proj_body,
        grid=(GRID,),
        in_specs=[
            pl.BlockSpec((BLK, d_in), lambda i: (i, 0)),
            pl.BlockSpec((d_in, H), lambda i: (0, 0)),
            pl.BlockSpec((1, H), lambda i: (0, 0)),
        ],
        out_specs=pl.BlockSpec((BLK, 128), lambda i: (i, 0)),
        out_shape=jax.ShapeDtypeStruct((NPAD, 128), jnp.float32),
    )(x, w, b.reshape(1, H))


def _sage_core(s, cnt, h, wl_ref, bl_ref, wr_ref, br_ref, g_ref, be_ref):
    agg = s / cnt
    out = (jnp.dot(agg, wl_ref[...], preferred_element_type=jnp.float32)
           + bl_ref[...]
           + jnp.dot(h, wr_ref[...], preferred_element_type=jnp.float32)
           + br_ref[...])
    nrm = jnp.sqrt(jnp.sum(out * out, axis=-1, keepdims=True))
    out = out / jnp.maximum(nrm, 1e-12)
    r = jnp.maximum(out, 0.0)
    m = jnp.mean(r, axis=-1, keepdims=True)
    v = jnp.mean((r - m) ** 2, axis=-1, keepdims=True)
    return (r - m) / jnp.sqrt(v + 1e-5) * g_ref[...] + be_ref[...]


def _update_body(s_ref, h_ref, wl_ref, bl_ref, wr_ref, br_ref, g_ref,
                 be_ref, o_ref):
    cnt = jnp.maximum(s_ref[:, H:H + 1], 1.0)
    hn = _sage_core(s_ref[:, :H], cnt, h_ref[:, :H], wl_ref, bl_ref,
                    wr_ref, br_ref, g_ref, be_ref)
    o_ref[:, :H] = hn
    o_ref[:, H:] = jnp.zeros((BLK, 128 - H), jnp.float32)


def _final_body(s1_ref, s0_ref, h_ref, wl_ref, bl_ref, wr_ref, br_ref,
                g_ref, be_ref, wc_ref, bc_ref, logits_ref):
    cnt = jnp.maximum(s0_ref[:, H:H + 1], 1.0)
    hn = _sage_core(s1_ref[:, :H], cnt, h_ref[:, :H], wl_ref, bl_ref,
                    wr_ref, br_ref, g_ref, be_ref)
    logits_ref[...] = (jnp.dot(hn, wc_ref[...],
                               preferred_element_type=jnp.float32)
                       + bc_ref[...])


_W_SPECS = [
    pl.BlockSpec((H, H), lambda i: (0, 0)),
    pl.BlockSpec((1, H), lambda i: (0, 0)),
    pl.BlockSpec((H, H), lambda i: (0, 0)),
    pl.BlockSpec((1, H), lambda i: (0, 0)),
    pl.BlockSpec((1, H), lambda i: (0, 0)),
    pl.BlockSpec((1, H), lambda i: (0, 0)),
]

_B128 = pl.BlockSpec((BLK, 128), lambda i: (i, 0))


def _update(s, h, wl, bl, wr, br, g, be):
    return pl.pallas_call(
        _update_body,
        grid=(GRID,),
        in_specs=[_B128, _B128] + _W_SPECS,
        out_specs=_B128,
        out_shape=jax.ShapeDtypeStruct((NPAD, 128), jnp.float32),
    )(s, h, wl, bl.reshape(1, -1), wr, br.reshape(1, -1),
      g.reshape(1, -1), be.reshape(1, -1))


def _final(s1, s0, h, wl, bl, wr, br, g, be, wc, bc):
    n_cls = wc.shape[1]
    return pl.pallas_call(
        _final_body,
        grid=(GRID,),
        in_specs=[_B128, _B128, _B128] + _W_SPECS + [
            pl.BlockSpec((H, n_cls), lambda i: (0, 0)),
            pl.BlockSpec((1, n_cls), lambda i: (0, 0)),
        ],
        out_specs=pl.BlockSpec((BLK, n_cls), lambda i: (i, 0)),
        out_shape=jax.ShapeDtypeStruct((N, n_cls), jnp.float32),
    )(s1, s0, h, wl, bl.reshape(1, -1), wr, br.reshape(1, -1),
      g.reshape(1, -1), be.reshape(1, -1), wc, bc.reshape(1, -1))


def kernel(x_stay, x_code, ei_sc, ei_cs, W_in_stay, b_in_stay, W_in_code,
           b_in_code, Wl0_sc, bl0_sc, Wr0_sc, br0_sc, Wl0_cs, bl0_cs, Wr0_cs,
           br0_cs, g0, be0, Wl1_sc, bl1_sc, Wr1_sc, br1_sc, Wl1_cs, bl1_cs,
           Wr1_cs, br1_cs, g1, be1, Wc, bc):
    src_sc = ei_sc[0].reshape(E // CH, CH)
    dst_sc = ei_sc[1].reshape(E // CH, CH)
    src_cs = ei_cs[0].reshape(E // CH, CH)
    dst_cs = ei_cs[1].reshape(E // CH, CH)

    hs = _proj(x_stay, W_in_stay, b_in_stay)
    hc = _proj(x_code, W_in_code, b_in_code)

    s_code, s_stay = _sc_layer0(src_sc, dst_sc, src_cs, dst_cs,
                                hs.reshape(8 * NPAD, Q),
                                hc.reshape(8 * NPAD, Q))

    hc1 = _update(s_code, hc, Wl0_sc, bl0_sc, Wr0_sc, br0_sc, g0, be0)
    hs1 = _update(s_stay, hs, Wl0_cs, bl0_cs, Wr0_cs, br0_cs, g0, be0)

    s_stay1 = _sc_layer1(src_cs, dst_cs, hc1.reshape(8 * NPAD, Q))

    return _final(s_stay1, s_stay, hs1, Wl1_cs, bl1_cs, Wr1_cs, br1_cs,
                  g1, be1, Wc, bc)
